# baseline (speedup 1.0000x reference)
import functools

import jax
import jax.numpy as jnp
from jax import lax
from jax.experimental import pallas as pl
from jax.experimental.pallas import tpu as pltpu
from jax.experimental.pallas import tpu_sc as plsc

B = 16384
F = 512
DG = 32
NV = 100000
NC = 2
NS = 16
NW = NC * NS
BPW = B // NW
CH = 32
NCHUNK = BPW // CH
SCH = 128
NSCHUNK = BPW // SCH


def _sc_feat_body(pi_hbm, ni_hbm, features, pf_out, nf_out,
                  pi_v, ni_v, pf0, pf1, nf0, nf1,
                  sg0, sg1, sw0, sw1):
    wid = lax.axis_index("c") * NS + lax.axis_index("s")
    base = wid * BPW
    pltpu.sync_copy(pi_hbm.at[pl.ds(base, BPW)], pi_v)
    pltpu.sync_copy(ni_hbm.at[pl.ds(base, BPW)], ni_v)

    pf = (pf0, pf1)
    nf = (nf0, nf1)
    sg = (sg0, sg1)
    sw = (sw0, sw1)

    def fire(c):
        slot = c % 2
        isl = pl.ds(c * CH, CH)
        g1 = pltpu.async_copy(features.at[pi_v.at[isl]], pf[slot], sg[slot])
        g2 = pltpu.async_copy(features.at[ni_v.at[isl]], nf[slot], sg[slot])
        return g1, g2

    def write(c):
        slot = c % 2
        osl = pl.ds(base + c * CH, CH)
        w1 = pltpu.make_async_copy(pf[slot], pf_out.at[osl], sw[slot])
        w2 = pltpu.make_async_copy(nf[slot], nf_out.at[osl], sw[slot])
        w1.start()
        w2.start()
        return w1, w2

    gathers = {0: fire(0)}
    writes = {}
    for c in range(NCHUNK):
        if c >= 2:
            for w in writes.pop(c - 2):
                w.wait()
        if c + 1 < NCHUNK:
            gathers[c + 1] = fire(c + 1)
        for g in gathers.pop(c):
            g.wait()
        writes[c] = write(c)
    for w in writes.pop(NCHUNK - 2):
        w.wait()
    for w in writes.pop(NCHUNK - 1):
        w.wait()


@functools.partial(
    pl.kernel,
    out_type=(
        jax.ShapeDtypeStruct((B, F), jnp.float32),
        jax.ShapeDtypeStruct((B, F), jnp.float32),
    ),
    mesh=plsc.VectorSubcoreMesh(core_axis_name="c", subcore_axis_name="s"),
    scratch_types=[
        pltpu.VMEM((BPW,), jnp.int32),
        pltpu.VMEM((BPW,), jnp.int32),
        pltpu.VMEM((CH, F), jnp.float32),
        pltpu.VMEM((CH, F), jnp.float32),
        pltpu.VMEM((CH, F), jnp.float32),
        pltpu.VMEM((CH, F), jnp.float32),
        pltpu.SemaphoreType.DMA,
        pltpu.SemaphoreType.DMA,
        pltpu.SemaphoreType.DMA,
        pltpu.SemaphoreType.DMA,
    ],
)
def _sc_feat(*refs):
    _sc_feat_body(*refs)


def _sc_tab_body(ui_hbm, pi_hbm, ni_hbm, tab,
                 tu_out, tp_out, tn_out,
                 ui_v, pi_v, ni_v, b0, b1, b2, b3, b4, b5,
                 sg0, sg1, sw0, sw1):
    wid = lax.axis_index("c") * NS + lax.axis_index("s")
    base = wid * BPW
    pltpu.sync_copy(ui_hbm.at[pl.ds(base, BPW)], ui_v)
    pltpu.sync_copy(pi_hbm.at[pl.ds(base, BPW)], pi_v)
    pltpu.sync_copy(ni_hbm.at[pl.ds(base, BPW)], ni_v)

    bufs = ((b0, b1, b2), (b3, b4, b5))
    sg = (sg0, sg1)
    sw = (sw0, sw1)
    idx = (ui_v, pi_v, ni_v)
    outs = (tu_out, tp_out, tn_out)

    def fire(c):
        slot = c % 2
        isl = pl.ds(c * SCH, SCH)
        return [pltpu.async_copy(tab.at[idx[t].at[isl]], bufs[slot][t], sg[slot])
                for t in range(3)]

    def write(c):
        slot = c % 2
        osl = pl.ds(base + c * SCH, SCH)
        ws = [pltpu.make_async_copy(bufs[slot][t], outs[t].at[osl], sw[slot])
              for t in range(3)]
        for w in ws:
            w.start()
        return ws

    gathers = {0: fire(0)}
    writes = {}
    for c in range(NSCHUNK):
        if c >= 2:
            for w in writes.pop(c - 2):
                w.wait()
        if c + 1 < NSCHUNK:
            gathers[c + 1] = fire(c + 1)
        for g in gathers.pop(c):
            g.wait()
        writes[c] = write(c)
    for w in writes.pop(NSCHUNK - 2):
        w.wait()
    for w in writes.pop(NSCHUNK - 1):
        w.wait()


@functools.partial(
    pl.kernel,
    out_type=(
        jax.ShapeDtypeStruct((B, 128), jnp.float32),
        jax.ShapeDtypeStruct((B, 128), jnp.float32),
        jax.ShapeDtypeStruct((B, 128), jnp.float32),
    ),
    mesh=plsc.VectorSubcoreMesh(core_axis_name="c", subcore_axis_name="s"),
    scratch_types=[
        pltpu.VMEM((BPW,), jnp.int32),
        pltpu.VMEM((BPW,), jnp.int32),
        pltpu.VMEM((BPW,), jnp.int32),
        pltpu.VMEM((SCH, 128), jnp.float32),
        pltpu.VMEM((SCH, 128), jnp.float32),
        pltpu.VMEM((SCH, 128), jnp.float32),
        pltpu.VMEM((SCH, 128), jnp.float32),
        pltpu.VMEM((SCH, 128), jnp.float32),
        pltpu.VMEM((SCH, 128), jnp.float32),
        pltpu.SemaphoreType.DMA,
        pltpu.SemaphoreType.DMA,
        pltpu.SemaphoreType.DMA,
        pltpu.SemaphoreType.DMA,
    ],
)
def _sc_tab(*refs):
    _sc_tab_body(*refs)


RP = 5000


def _tc_prep_body(gu, tu, gi, bi, tab):
    tab[...] = jnp.concatenate(
        [gu[...], tu[...], gi[...], bi[...],
         jnp.zeros((RP, 128 - 3 * DG - 1), jnp.float32)], axis=1)


def _tc_prep(gamma_users, theta_users, gamma_items, beta_items):
    bs = pl.BlockSpec((RP, DG), lambda i: (i, 0))
    return pl.pallas_call(
        _tc_prep_body,
        grid=(NV // RP,),
        in_specs=[bs, bs, bs, pl.BlockSpec((RP, 1), lambda i: (i, 0))],
        out_specs=pl.BlockSpec((RP, 128), lambda i: (i, 0)),
        out_shape=jax.ShapeDtypeStruct((NV, 128), jnp.float32),
    )(gamma_users, theta_users, gamma_items, beta_items)


BB = 1024


def _tc_combine_body(pf, nf, tabu, tabp, tabn, emb, vb, out):
    gu = tabu[:, 0:DG]
    tu = tabu[:, DG:2 * DG]
    gip = tabp[:, 2 * DG:3 * DG]
    gin = tabn[:, 2 * DG:3 * DG]
    bp = tabp[:, 3 * DG:3 * DG + 1]
    bn = tabn[:, 3 * DG:3 * DG + 1]
    diff = pf[...] - nf[...]
    g = jnp.dot(diff, emb[...], preferred_element_type=jnp.float32,
                precision=lax.Precision.HIGHEST)
    s_vis = jnp.sum(tu * g, axis=1, keepdims=True)
    s_bias = jnp.dot(diff, vb[...], preferred_element_type=jnp.float32,
                     precision=lax.Precision.HIGHEST)
    s_lat = jnp.sum(gu * (gip - gin), axis=1, keepdims=True)
    out[...] = bp - bn + s_lat + s_vis + s_bias


def _tc_combine(pf, nf, tabu, tabp, tabn, emb, vb):
    bspec_f = pl.BlockSpec((BB, F), lambda i: (i, 0))
    bspec_s = pl.BlockSpec((BB, 128), lambda i: (i, 0))
    return pl.pallas_call(
        _tc_combine_body,
        grid=(B // BB,),
        in_specs=[
            bspec_f, bspec_f, bspec_s, bspec_s, bspec_s,
            pl.BlockSpec((F, DG), lambda i: (0, 0)),
            pl.BlockSpec((F, 1), lambda i: (0, 0)),
        ],
        out_specs=pl.BlockSpec((BB, 1), lambda i: (i, 0)),
        out_shape=jax.ShapeDtypeStruct((B, 1), jnp.float32),
    )(pf, nf, tabu, tabp, tabn, emb, vb)[:, 0]


def kernel(ui, pi, ni, features, gamma_users, gamma_items, theta_users,
           embedding, beta_items, visual_bias):
    tab = _tc_prep(gamma_users, theta_users, gamma_items, beta_items)
    pf, nf = _sc_feat(pi, ni, features)
    tabu, tabp, tabn = _sc_tab(ui, pi, ni, tab)
    return _tc_combine(pf, nf, tabu, tabp, tabn, embedding, visual_bias)
